# Initial kernel scaffold; baseline (speedup 1.0000x reference)
import functools

import jax
import jax.numpy as jnp
from jax.experimental import pallas as pl
from jax.experimental.pallas import tpu as pltpu
from jax.experimental.pallas import tpu_sc as plsc

N = 10000
E = 320000
F = 128
EDIM = 16
H = 32

NC = 2
NS = 16
NW = NC * NS
EPW = E // NW
CHUNK = 125
NCHUNK = EPW // CHUNK


def _node_mlp_kernel(x_ref, w1a_ref, b1_ref, u_ref):
    u_ref[...] = jnp.dot(x_ref[...], w1a_ref[...],
                         preferred_element_type=jnp.float32) + b1_ref[...]


def _edge_mlp_kernel(g_ref, ea_ref, w1b_ref, w2_ref, b2_ref, m_ref):
    h = jax.nn.relu(g_ref[...] + jnp.dot(ea_ref[...], w1b_ref[...],
                                         preferred_element_type=jnp.float32))
    m_ref[...] = jax.nn.relu(jnp.dot(h, w2_ref[...],
                                     preferred_element_type=jnp.float32)
                             + b2_ref[...])


def _combine_kernel(p_ref, cnt_ref, x_ref, o_ref):
    cnt = jnp.sum(cnt_ref[...], axis=0)
    s = p_ref[0] + p_ref[1]
    inv = 1.0 / jnp.maximum(cnt, 1.0)
    o_ref[...] = s * inv.reshape(-1, 1) + x_ref[...]


def kernel(x, edge_index, edge_attr, W1, b1, W2, b2):
    dst = edge_index[1]
    w1a = W1[:F]
    w1b = W1[F:]

    u = pl.pallas_call(
        _node_mlp_kernel,
        out_shape=jax.ShapeDtypeStruct((N, H), jnp.float32),
    )(x, w1a, b1)

    g = jnp.take(u, dst, axis=0)
    cnt32 = jax.ops.segment_sum(
        jnp.ones((E,), jnp.float32), dst, num_segments=N).reshape(1, N)

    TE = 2000
    m = pl.pallas_call(
        _edge_mlp_kernel,
        grid=(E // TE,),
        in_specs=[
            pl.BlockSpec((TE, H), lambda i: (i, 0)),
            pl.BlockSpec((TE, EDIM), lambda i: (i, 0)),
            pl.BlockSpec((EDIM, H), lambda i: (0, 0)),
            pl.BlockSpec((H, F), lambda i: (0, 0)),
            pl.BlockSpec((1, F), lambda i: (0, 0)),
        ],
        out_specs=pl.BlockSpec((TE, F), lambda i: (i, 0)),
        out_shape=jax.ShapeDtypeStruct((E, F), jnp.float32),
    )(g, edge_attr, w1b, W2, b2.reshape(1, F))

    p0 = jax.ops.segment_sum(m[: E // 2], dst[: E // 2], num_segments=N)
    p1 = jax.ops.segment_sum(m[E // 2:], dst[E // 2:], num_segments=N)
    partials = jnp.stack([p0, p1])

    out = pl.pallas_call(
        _combine_kernel,
        out_shape=jax.ShapeDtypeStruct((N, F), jnp.float32),
    )(partials, cnt32, x)
    return out


# speedup vs baseline: 3.7271x; 3.7271x over previous
import dataclasses
import functools

import jax
import jax.numpy as jnp
from jax.experimental import pallas as pl
from jax.experimental.pallas import tpu as pltpu
from jax.experimental.pallas import tpu_sc as plsc

N = 10000
E = 320000
F = 128
EDIM = 16
H = 32

NC = 2
NS = 16
NW = NC * NS
EPW = E // NW
CHUNK = 125
NCHUNK = EPW // CHUNK


def _node_mlp_kernel(x_ref, w1a_ref, b1_ref, u_ref):
    u_ref[...] = jnp.dot(x_ref[...], w1a_ref[...],
                         preferred_element_type=jnp.float32) + b1_ref[...]


def _edge_mlp_kernel(g_ref, ea_ref, w1b_ref, w2_ref, b2_ref, m_ref):
    h = jax.nn.relu(g_ref[...] + jnp.dot(ea_ref[...], w1b_ref[...],
                                         preferred_element_type=jnp.float32))
    m_ref[...] = jax.nn.relu(jnp.dot(h, w2_ref[...],
                                     preferred_element_type=jnp.float32)
                             + b2_ref[...])


def _combine_kernel(p_ref, cnt_ref, x_ref, o_ref):
    cnt = jnp.sum(cnt_ref[...], axis=0)
    s = p_ref[0] + p_ref[1]
    inv = 1.0 / jnp.maximum(cnt, 1.0)
    o_ref[...] = s * inv.reshape(-1, 1) + x_ref[...]


_SC_MESH = plsc.VectorSubcoreMesh(core_axis_name="c", subcore_axis_name="s")
_SC_PARAMS = pltpu.CompilerParams(needs_layout_passes=False,
                                  use_tc_tiling_on_sc=False)


def _gather_cnt_body(u_hbm, dstg_hbm, dstf_hbm, g_hbm, cnt_hbm,
                     idx_v, cnt_v, sem):
    c = jax.lax.axis_index("c")
    s = jax.lax.axis_index("s")
    wid = s * NC + c

    def body(i_vmem, o_vmem):
        pltpu.sync_copy(u_hbm.at[i_vmem.at[0, 0]], o_vmem.at[0])

    pltpu.emit_pipeline(
        body,
        grid=(E // CHUNK,),
        in_specs=[pl.BlockSpec((1, 1, CHUNK), lambda i: (i, 0, 0))],
        out_specs=[pl.BlockSpec((1, CHUNK, H), lambda i: (i, 0, 0))],
        core_axis_name=("c", "s"),
        dimension_semantics=(pltpu.PARALLEL,),
    )(dstg_hbm, g_hbm)

    pltpu.async_copy(dstf_hbm.at[wid, 0], idx_v, sem).wait()

    @pl.loop(0, N, step=16)
    def _zero(i):
        cnt_v[pl.ds(i, 16)] = jnp.zeros((16,), jnp.float32)

    ones = jnp.ones((16,), jnp.float32)

    @pl.loop(0, EPW, step=16)
    def _hist(i):
        idx = idx_v[pl.ds(i, 16)]
        plsc.addupdate_scatter(cnt_v, [idx], ones)

    pltpu.sync_copy(cnt_v, cnt_hbm.at[wid, 0])


def _scatter_body(m_hbm, dst3_hbm, p_hbm, idx_v, buf_v, acc_sh, sem):
    c = jax.lax.axis_index("c")
    s = jax.lax.axis_index("s")
    wid = s * NC + c
    cbase = wid * NCHUNK

    @pl.loop(0, CHUNK)
    def _zr(i):
        @pl.loop(0, F, step=16)
        def _zc(j):
            buf_v[i, pl.ds(j, 16)] = jnp.zeros((16,), jnp.float32)

    @pl.loop(0, N // NS, step=CHUNK)
    def _zacc(r):
        pltpu.sync_copy(buf_v, acc_sh.at[pl.ds(s * (N // NS) + r, CHUNK)])

    plsc.subcore_barrier()

    pltpu.async_copy(dst3_hbm.at[wid], idx_v, sem).wait()

    @pl.loop(0, NCHUNK)
    def _chunk(j):
        pltpu.sync_copy(m_hbm.at[cbase + j], buf_v)
        pltpu.sync_copy(buf_v, acc_sh.at[idx_v.at[j]], add=True)

    plsc.subcore_barrier()

    pltpu.sync_copy(acc_sh.at[pl.ds(s * (N // NS), N // NS)],
                    p_hbm.at[c, pl.ds(s * (N // NS), N // NS)])


@functools.partial(
    pl.kernel,
    out_type=[jax.ShapeDtypeStruct((E // CHUNK, CHUNK, H), jnp.float32),
              jax.ShapeDtypeStruct((NW, 1, N), jnp.float32)],
    mesh=_SC_MESH,
    compiler_params=_SC_PARAMS,
    scratch_types=[
        pltpu.VMEM((EPW,), jnp.int32),
        pltpu.VMEM((N,), jnp.float32),
        pltpu.SemaphoreType.DMA,
    ],
)
def _gather_cnt(u_hbm, dstg_hbm, dstf_hbm, g_hbm, cnt_hbm, idx_v, cnt_v, sem):
    _gather_cnt_body(u_hbm, dstg_hbm, dstf_hbm, g_hbm, cnt_hbm,
                     idx_v, cnt_v, sem)


@functools.partial(
    pl.kernel,
    out_type=jax.ShapeDtypeStruct((NC, N, F), jnp.float32),
    mesh=_SC_MESH,
    compiler_params=_SC_PARAMS,
    scratch_types=[
        pltpu.VMEM((NCHUNK, CHUNK), jnp.int32),
        pltpu.VMEM((CHUNK, F), jnp.float32),
        pltpu.VMEM_SHARED((N, F), jnp.float32),
        pltpu.SemaphoreType.DMA,
    ],
)
def _scatter(m_hbm, dst3_hbm, p_hbm, idx_v, buf_v, acc_sh, sem):
    _scatter_body(m_hbm, dst3_hbm, p_hbm, idx_v, buf_v, acc_sh, sem)


def kernel(x, edge_index, edge_attr, W1, b1, W2, b2):
    dst = edge_index[1]
    w1a = W1[:F]
    w1b = W1[F:]

    u = pl.pallas_call(
        _node_mlp_kernel,
        out_shape=jax.ShapeDtypeStruct((N, H), jnp.float32),
    )(x, w1a, b1)

    g3, cnt_t3 = _gather_cnt(u, dst.reshape(E // CHUNK, 1, CHUNK),
                             dst.reshape(NW, 1, EPW))
    g = g3.reshape(E, H)
    cnt_t = cnt_t3.reshape(NW, N)

    TE = 2000
    m = pl.pallas_call(
        _edge_mlp_kernel,
        grid=(E // TE,),
        in_specs=[
            pl.BlockSpec((TE, H), lambda i: (i, 0)),
            pl.BlockSpec((TE, EDIM), lambda i: (i, 0)),
            pl.BlockSpec((EDIM, H), lambda i: (0, 0)),
            pl.BlockSpec((H, F), lambda i: (0, 0)),
            pl.BlockSpec((1, F), lambda i: (0, 0)),
        ],
        out_specs=pl.BlockSpec((TE, F), lambda i: (i, 0)),
        out_shape=jax.ShapeDtypeStruct((E, F), jnp.float32),
    )(g, edge_attr, w1b, W2, b2.reshape(1, F))

    partials = _scatter(m.reshape(E // CHUNK, CHUNK, F),
                        dst.reshape(NW, NCHUNK, CHUNK))

    out = pl.pallas_call(
        _combine_kernel,
        out_shape=jax.ShapeDtypeStruct((N, F), jnp.float32),
    )(partials, cnt_t, x)
    return out


# speedup vs baseline: 3.7292x; 1.0006x over previous
import dataclasses
import functools

import jax
import jax.numpy as jnp
from jax.experimental import pallas as pl
from jax.experimental.pallas import tpu as pltpu
from jax.experimental.pallas import tpu_sc as plsc

N = 10000
E = 320000
F = 128
EDIM = 16
H = 32

NC = 2
NS = 16
NW = NC * NS
EPW = E // NW
CHUNK = 125
NCHUNK = EPW // CHUNK


def _node_mlp_kernel(x_ref, w1a_ref, b1_ref, u_ref):
    u_ref[...] = jnp.dot(x_ref[...], w1a_ref[...],
                         preferred_element_type=jnp.float32) + b1_ref[...]


def _edge_mlp_kernel(g_ref, ea_ref, w1b_ref, w2_ref, b2_ref, m_ref):
    h = jax.nn.relu(g_ref[...] + jnp.dot(ea_ref[...], w1b_ref[...],
                                         preferred_element_type=jnp.float32))
    m_ref[...] = jax.nn.relu(jnp.dot(h, w2_ref[...],
                                     preferred_element_type=jnp.float32)
                             + b2_ref[...])


def _combine_kernel(p_ref, cnt_ref, x_ref, o_ref):
    cnt = jnp.sum(cnt_ref[...], axis=0)
    s = p_ref[0] + p_ref[1]
    inv = 1.0 / jnp.maximum(cnt, 1.0)
    o_ref[...] = s * inv.reshape(-1, 1) + x_ref[...]


_SC_MESH = plsc.VectorSubcoreMesh(core_axis_name="c", subcore_axis_name="s")
_SC_PARAMS = pltpu.CompilerParams(needs_layout_passes=False,
                                  use_tc_tiling_on_sc=False)


def _gather_cnt_body(u_hbm, dstg_hbm, dstf_hbm, g_hbm, cnt_hbm,
                     idx_v, cnt_v, sem):
    c = jax.lax.axis_index("c")
    s = jax.lax.axis_index("s")
    wid = s * NC + c

    def body(i_vmem, o_vmem):
        pltpu.sync_copy(u_hbm.at[i_vmem.at[0, 0]], o_vmem)

    pltpu.emit_pipeline(
        body,
        grid=(E // CHUNK,),
        in_specs=[pl.BlockSpec((1, 1, CHUNK), lambda i: (i, 0, 0))],
        out_specs=[pl.BlockSpec((CHUNK, H), lambda i: (i, 0))],
        core_axis_name=("c", "s"),
        dimension_semantics=(pltpu.PARALLEL,),
    )(dstg_hbm, g_hbm)

    pltpu.async_copy(dstf_hbm.at[wid, 0], idx_v, sem).wait()

    @pl.loop(0, N, step=16)
    def _zero(i):
        cnt_v[pl.ds(i, 16)] = jnp.zeros((16,), jnp.float32)

    ones = jnp.ones((16,), jnp.float32)

    @pl.loop(0, EPW, step=16)
    def _hist(i):
        idx = idx_v[pl.ds(i, 16)]
        plsc.addupdate_scatter(cnt_v, [idx], ones)

    pltpu.sync_copy(cnt_v, cnt_hbm.at[wid, 0])


def _scatter_body(m_hbm, dst3_hbm, p_hbm, idx_v, buf_v, acc_sh, sem):
    c = jax.lax.axis_index("c")
    s = jax.lax.axis_index("s")
    wid = s * NC + c
    ebase = wid * EPW

    @pl.loop(0, CHUNK)
    def _zr(i):
        @pl.loop(0, F, step=16)
        def _zc(j):
            buf_v[i, pl.ds(j, 16)] = jnp.zeros((16,), jnp.float32)

    @pl.loop(0, N // NS, step=CHUNK)
    def _zacc(r):
        pltpu.sync_copy(buf_v, acc_sh.at[pl.ds(s * (N // NS) + r, CHUNK)])

    plsc.subcore_barrier()

    pltpu.async_copy(dst3_hbm.at[wid], idx_v, sem).wait()

    @pl.loop(0, NCHUNK)
    def _chunk(j):
        pltpu.sync_copy(m_hbm.at[pl.ds(ebase + j * CHUNK, CHUNK)], buf_v)
        pltpu.sync_copy(buf_v, acc_sh.at[idx_v.at[j]], add=True)

    plsc.subcore_barrier()

    pltpu.sync_copy(acc_sh.at[pl.ds(s * (N // NS), N // NS)],
                    p_hbm.at[c, pl.ds(s * (N // NS), N // NS)])


@functools.partial(
    pl.kernel,
    out_type=[jax.ShapeDtypeStruct((E, H), jnp.float32),
              jax.ShapeDtypeStruct((NW, 1, N), jnp.float32)],
    mesh=_SC_MESH,
    compiler_params=_SC_PARAMS,
    scratch_types=[
        pltpu.VMEM((EPW,), jnp.int32),
        pltpu.VMEM((N,), jnp.float32),
        pltpu.SemaphoreType.DMA,
    ],
)
def _gather_cnt(u_hbm, dstg_hbm, dstf_hbm, g_hbm, cnt_hbm, idx_v, cnt_v, sem):
    _gather_cnt_body(u_hbm, dstg_hbm, dstf_hbm, g_hbm, cnt_hbm,
                     idx_v, cnt_v, sem)


@functools.partial(
    pl.kernel,
    out_type=jax.ShapeDtypeStruct((NC, N, F), jnp.float32),
    mesh=_SC_MESH,
    compiler_params=_SC_PARAMS,
    scratch_types=[
        pltpu.VMEM((NCHUNK, CHUNK), jnp.int32),
        pltpu.VMEM((CHUNK, F), jnp.float32),
        pltpu.VMEM_SHARED((N, F), jnp.float32),
        pltpu.SemaphoreType.DMA,
    ],
)
def _scatter(m_hbm, dst3_hbm, p_hbm, idx_v, buf_v, acc_sh, sem):
    _scatter_body(m_hbm, dst3_hbm, p_hbm, idx_v, buf_v, acc_sh, sem)


def kernel(x, edge_index, edge_attr, W1, b1, W2, b2):
    dst = edge_index[1]
    w1a = W1[:F]
    w1b = W1[F:]

    u = pl.pallas_call(
        _node_mlp_kernel,
        out_shape=jax.ShapeDtypeStruct((N, H), jnp.float32),
    )(x, w1a, b1)

    g, cnt_t3 = _gather_cnt(u, dst.reshape(E // CHUNK, 1, CHUNK),
                            dst.reshape(NW, 1, EPW))
    cnt_t = cnt_t3.reshape(NW, N)

    TE = 2000
    m = pl.pallas_call(
        _edge_mlp_kernel,
        grid=(E // TE,),
        in_specs=[
            pl.BlockSpec((TE, H), lambda i: (i, 0)),
            pl.BlockSpec((TE, EDIM), lambda i: (i, 0)),
            pl.BlockSpec((EDIM, H), lambda i: (0, 0)),
            pl.BlockSpec((H, F), lambda i: (0, 0)),
            pl.BlockSpec((1, F), lambda i: (0, 0)),
        ],
        out_specs=pl.BlockSpec((TE, F), lambda i: (i, 0)),
        out_shape=jax.ShapeDtypeStruct((E, F), jnp.float32),
    )(g, edge_attr, w1b, W2, b2.reshape(1, F))

    partials = _scatter(m, dst.reshape(NW, NCHUNK, CHUNK))

    out = pl.pallas_call(
        _combine_kernel,
        out_shape=jax.ShapeDtypeStruct((N, F), jnp.float32),
    )(partials, cnt_t, x)
    return out


# speedup vs baseline: 4.4817x; 1.2018x over previous
import dataclasses
import functools

import jax
import jax.numpy as jnp
from jax.experimental import pallas as pl
from jax.experimental.pallas import tpu as pltpu
from jax.experimental.pallas import tpu_sc as plsc

N = 10000
E = 320000
F = 128
EDIM = 16
H = 32

NC = 2
NS = 16
NW = NC * NS
EPW = E // NW
CHUNK = 125
NCHUNK = EPW // CHUNK


def _node_mlp_kernel(x_ref, w1a_ref, b1_ref, u_ref):
    u_ref[...] = jnp.dot(x_ref[...], w1a_ref[...],
                         preferred_element_type=jnp.float32) + b1_ref[...]


def _edge_mlp_kernel(g4_ref, ea4_ref, w1bd_ref, w2k_ref, b2_ref,
                     m0_ref, m1_ref, m2_ref, m3_ref):
    pre = jnp.dot(ea4_ref[...], w1bd_ref[...],
                  preferred_element_type=jnp.float32)
    h4 = jax.nn.relu(g4_ref[...] + pre)
    outs = (m0_ref, m1_ref, m2_ref, m3_ref)
    for k in range(4):
        mk = jnp.dot(h4, w2k_ref[k], preferred_element_type=jnp.float32)
        outs[k][...] = jax.nn.relu(mk + b2_ref[...])


def _combine_kernel(p_ref, cnt_ref, x_ref, o_ref):
    cnt = jnp.sum(cnt_ref[...], axis=0)
    s = p_ref[0] + p_ref[1]
    inv = 1.0 / jnp.maximum(cnt, 1.0)
    o_ref[...] = s * inv.reshape(-1, 1) + x_ref[...]


_SC_MESH = plsc.VectorSubcoreMesh(core_axis_name="c", subcore_axis_name="s")
_SC_PARAMS = pltpu.CompilerParams(needs_layout_passes=False,
                                  use_tc_tiling_on_sc=False)


def _gather_cnt_body(u_hbm, dstg_hbm, dstf_hbm, g_hbm, cnt_hbm,
                     idx_v, cnt_v, sem):
    c = jax.lax.axis_index("c")
    s = jax.lax.axis_index("s")
    wid = s * NC + c

    def body(i_vmem, o_vmem):
        pltpu.sync_copy(u_hbm.at[i_vmem.at[0, 0]], o_vmem)

    pltpu.emit_pipeline(
        body,
        grid=(E // CHUNK,),
        in_specs=[pl.BlockSpec((1, 1, CHUNK), lambda i: (i, 0, 0))],
        out_specs=[pl.BlockSpec((CHUNK, H), lambda i: (i, 0))],
        core_axis_name=("c", "s"),
        dimension_semantics=(pltpu.PARALLEL,),
    )(dstg_hbm, g_hbm)

    pltpu.async_copy(dstf_hbm.at[wid, 0], idx_v, sem).wait()

    @pl.loop(0, N, step=16)
    def _zero(i):
        cnt_v[pl.ds(i, 16)] = jnp.zeros((16,), jnp.float32)

    ones = jnp.ones((16,), jnp.float32)

    @pl.loop(0, EPW, step=16)
    def _hist(i):
        idx = idx_v[pl.ds(i, 16)]
        plsc.addupdate_scatter(cnt_v, [idx], ones)

    pltpu.sync_copy(cnt_v, cnt_hbm.at[wid, 0])


NQT = 8
QCH = (E // 4) // CHUNK
CPT = QCH // NQT


def _scatter_body(m0_hbm, m1_hbm, m2_hbm, m3_hbm, dst4_hbm, p_hbm,
                  idx_v, buf_v, acc_sh, sem):
    c = jax.lax.axis_index("c")
    s = jax.lax.axis_index("s")
    wid = s * NC + c
    q = wid // NQT
    t = wid % NQT

    @pl.loop(0, CHUNK)
    def _zr(i):
        @pl.loop(0, F, step=16)
        def _zc(j):
            buf_v[i, pl.ds(j, 16)] = jnp.zeros((16,), jnp.float32)

    @pl.loop(0, N // NS, step=CHUNK)
    def _zacc(r):
        pltpu.sync_copy(buf_v, acc_sh.at[pl.ds(s * (N // NS) + r, CHUNK)])

    plsc.subcore_barrier()

    pltpu.async_copy(dst4_hbm.at[q, pl.ds(t * CPT, CPT)], idx_v, sem).wait()

    for k, mk_hbm in enumerate((m0_hbm, m1_hbm, m2_hbm, m3_hbm)):
        @pl.when(q == k)
        def _quarter(mk_hbm=mk_hbm):
            @pl.loop(0, CPT)
            def _chunk(j):
                pltpu.sync_copy(
                    mk_hbm.at[pl.ds((t * CPT + j) * CHUNK, CHUNK)], buf_v)
                pltpu.sync_copy(buf_v, acc_sh.at[idx_v.at[j]], add=True)

    plsc.subcore_barrier()

    pltpu.sync_copy(acc_sh.at[pl.ds(s * (N // NS), N // NS)],
                    p_hbm.at[c, pl.ds(s * (N // NS), N // NS)])


@functools.partial(
    pl.kernel,
    out_type=[jax.ShapeDtypeStruct((E, H), jnp.float32),
              jax.ShapeDtypeStruct((NW, 1, N), jnp.float32)],
    mesh=_SC_MESH,
    compiler_params=_SC_PARAMS,
    scratch_types=[
        pltpu.VMEM((EPW,), jnp.int32),
        pltpu.VMEM((N,), jnp.float32),
        pltpu.SemaphoreType.DMA,
    ],
)
def _gather_cnt(u_hbm, dstg_hbm, dstf_hbm, g_hbm, cnt_hbm, idx_v, cnt_v, sem):
    _gather_cnt_body(u_hbm, dstg_hbm, dstf_hbm, g_hbm, cnt_hbm,
                     idx_v, cnt_v, sem)


@functools.partial(
    pl.kernel,
    out_type=jax.ShapeDtypeStruct((NC, N, F), jnp.float32),
    mesh=_SC_MESH,
    compiler_params=_SC_PARAMS,
    scratch_types=[
        pltpu.VMEM((CPT, CHUNK), jnp.int32),
        pltpu.VMEM((CHUNK, F), jnp.float32),
        pltpu.VMEM_SHARED((N, F), jnp.float32),
        pltpu.SemaphoreType.DMA,
    ],
)
def _scatter(m0_hbm, m1_hbm, m2_hbm, m3_hbm, dst4_hbm, p_hbm,
             idx_v, buf_v, acc_sh, sem):
    _scatter_body(m0_hbm, m1_hbm, m2_hbm, m3_hbm, dst4_hbm, p_hbm,
                  idx_v, buf_v, acc_sh, sem)


def kernel(x, edge_index, edge_attr, W1, b1, W2, b2):
    dst = edge_index[1]
    w1a = W1[:F]
    w1b = W1[F:]

    u = pl.pallas_call(
        _node_mlp_kernel,
        out_shape=jax.ShapeDtypeStruct((N, H), jnp.float32),
    )(x, w1a, b1)

    g, cnt_t3 = _gather_cnt(u, dst.reshape(E // CHUNK, 1, CHUNK),
                            dst.reshape(NW, 1, EPW))
    cnt_t = cnt_t3.reshape(NW, N)

    E4 = E // 4
    g4 = g.reshape(E4, F)
    ea4 = edge_attr.reshape(E4, 4 * EDIM)
    w1bd = jax.scipy.linalg.block_diag(w1b, w1b, w1b, w1b)
    w2k = jnp.stack([jnp.pad(W2, ((k * H, F - H - k * H), (0, 0)))
                     for k in range(4)])
    B4 = 1000
    mq = pl.pallas_call(
        _edge_mlp_kernel,
        grid=(E4 // B4,),
        in_specs=[
            pl.BlockSpec((B4, F), lambda i: (i, 0)),
            pl.BlockSpec((B4, 4 * EDIM), lambda i: (i, 0)),
            pl.BlockSpec((4 * EDIM, F), lambda i: (0, 0)),
            pl.BlockSpec((4, F, F), lambda i: (0, 0, 0)),
            pl.BlockSpec((1, F), lambda i: (0, 0)),
        ],
        out_specs=[pl.BlockSpec((B4, F), lambda i: (i, 0))] * 4,
        out_shape=[jax.ShapeDtypeStruct((E4, F), jnp.float32)] * 4,
    )(g4, ea4, w1bd, w2k, b2.reshape(1, F))

    dst4 = dst.reshape(E4, 4).T.reshape(4, QCH, CHUNK)
    partials = _scatter(*mq, dst4)

    out = pl.pallas_call(
        _combine_kernel,
        out_shape=jax.ShapeDtypeStruct((N, F), jnp.float32),
    )(partials, cnt_t, x)
    return out
